# baseline (speedup 1.0000x reference)
import jax
import jax.numpy as jnp
from jax import lax
from jax.experimental import pallas as pl
from jax.experimental.pallas import tpu as pltpu
from jax.experimental.pallas import tpu_sc as plsc

B = 16
CLS = 100
POOL = 100
CTX_LEN = 12
D = 512
SEQ = 77
SUF = SEQ - 1 - CTX_LEN * 2
NC_SUF = SEQ - 1 - CTX_LEN
NC_SEQ = 1 + 2 * CTX_LEN + NC_SUF

CB = 100
NCB = CLS // CB


def _sc_gather_body(ig, ia, gp, ap, out, idx_v, rows_v, sem):
    c = lax.axis_index("c")
    s = lax.axis_index("s")
    wid = s * 2 + c

    @pl.when(wid == 0)
    def _():
        pltpu.sync_copy(ig, idx_v)
        pltpu.async_copy(gp.at[idx_v], rows_v, sem).wait()
        cps = [pltpu.async_copy(rows_v.at[i], out.at[i // 2, i % 2], sem)
               for i in range(B)]
        for cp in cps:
            cp.wait()

    @pl.when(wid == 1)
    def _():
        pltpu.sync_copy(ia, idx_v)
        pltpu.async_copy(ap.at[idx_v], rows_v, sem).wait()
        cps = [pltpu.async_copy(rows_v.at[i], out.at[8 + i // 2, i % 2], sem)
               for i in range(B)]
        for cp in cps:
            cp.wait()


def _sc_gather(ig, ia, gp, ap):
    return pl.kernel(
        _sc_gather_body,
        out_type=jax.ShapeDtypeStruct((B, 2, CTX_LEN, D), jnp.float32),
        mesh=plsc.VectorSubcoreMesh(core_axis_name="c", subcore_axis_name="s"),
        compiler_params=pltpu.CompilerParams(use_tc_tiling_on_sc=False),
        scratch_types=[
            pltpu.VMEM((B,), jnp.int32),
            pltpu.VMEM((B, CTX_LEN, D), jnp.float32),
            pltpu.SemaphoreType.DMA,
        ],
    )(ig, ia, gp, ap)


NBUF = 2
NSTEP = B * NCB


def _assemble(buf, ctx, pre, suf, cb):
    buf[:, 0:1, :] = pre[...]
    buf[:, 1:1 + CTX_LEN, :] = jnp.broadcast_to(ctx[0, 0], (CB, CTX_LEN, D))
    buf[:, 1 + CTX_LEN:1 + 2 * CTX_LEN, :] = jnp.broadcast_to(
        ctx[0, 1], (CB, CTX_LEN, D))
    buf[:, 1 + 2 * CTX_LEN:SEQ, :] = suf[pl.ds(cb * CB, CB)]


def _prompts_body(ctx, pre, suf, out, bufs, sems):
    s = pl.program_id(0)
    cb = s % NCB
    i = s % NBUF

    @pl.when(s >= NBUF)
    def _():
        pltpu.make_async_copy(
            bufs.at[i], out.at[pl.ds((s - NBUF) * CB, CB)], sems.at[i]
        ).wait()

    _assemble(bufs.at[i], ctx, pre, suf, cb)
    pltpu.make_async_copy(
        bufs.at[i], out.at[pl.ds(s * CB, CB)], sems.at[i]).start()

    @pl.when(s == NSTEP - 1)
    def _():
        for k in range(NBUF):
            t = NSTEP - NBUF + k
            pltpu.make_async_copy(
                bufs.at[t % NBUF], out.at[pl.ds(t * CB, CB)],
                sems.at[t % NBUF]).wait()


def _build_prompts_call():
    return pl.pallas_call(
        _prompts_body,
        grid=(NSTEP,),
        in_specs=[
            pl.BlockSpec((1, 2, CTX_LEN, D), lambda s: (s // NCB, 0, 0, 0)),
            pl.BlockSpec((CB, 1, D), lambda s: (s % NCB, 0, 0)),
            pl.BlockSpec((CLS, SUF, D), lambda s: (0, 0, 0)),
        ],
        out_specs=pl.BlockSpec(memory_space=pltpu.MemorySpace.HBM),
        out_shape=jax.ShapeDtypeStruct((B * CLS, SEQ, D), jnp.float32),
        scratch_shapes=[
            pltpu.VMEM((NBUF, CB, SEQ, D), jnp.float32),
            pltpu.SemaphoreType.DMA((NBUF,)),
        ],
    )


def _nc_body(ncp, gp, ap, ncs, nctok, tokp, out, nc_tok_out, tok_out):
    out[:, 0:1, :] = jnp.broadcast_to(ncp[...], (CB, 1, D))
    out[:, 1:1 + CTX_LEN, :] = gp[...]
    out[:, 1 + CTX_LEN:1 + 2 * CTX_LEN, :] = ap[...]
    out[:, 1 + 2 * CTX_LEN:NC_SEQ, :] = jnp.broadcast_to(
        ncs[...], (CB, NC_SUF, D))
    nc_tok_out[...] = jnp.broadcast_to(nctok[...], (POOL, SEQ))
    t = tokp[...]
    for b in range(B):
        tok_out[pl.ds(b * CLS, CLS), :] = t


def _build_nc_call():
    return pl.pallas_call(
        _nc_body,
        grid=(NCB,),
        in_specs=[
            pl.BlockSpec((1, 1, D), lambda i: (0, 0, 0)),
            pl.BlockSpec((CB, CTX_LEN, D), lambda i: (i, 0, 0)),
            pl.BlockSpec((CB, CTX_LEN, D), lambda i: (i, 0, 0)),
            pl.BlockSpec((1, NC_SUF, D), lambda i: (0, 0, 0)),
            pl.BlockSpec((1, SEQ), lambda i: (0, 0)),
            pl.BlockSpec((CLS, SEQ), lambda i: (0, 0)),
        ],
        out_specs=[
            pl.BlockSpec((CB, NC_SEQ, D), lambda i: (i, 0, 0)),
            pl.BlockSpec((POOL, SEQ), lambda i: (0, 0)),
            pl.BlockSpec((B * CLS, SEQ), lambda i: (0, 0)),
        ],
        out_shape=[
            jax.ShapeDtypeStruct((POOL, NC_SEQ, D), jnp.float32),
            jax.ShapeDtypeStruct((POOL, SEQ), jnp.int32),
            jax.ShapeDtypeStruct((B * CLS, SEQ), jnp.int32),
        ],
    )


def kernel(indices_g, indices_a, global_prompt, attribute_prompt,
           token_prefix, token_suffix, nc_token_prefix, nc_token_suffix,
           tokenized_prompts, nc_tokenized_prompts):
    ig = indices_g.astype(jnp.int32)
    ia = indices_a.astype(jnp.int32)
    tokp = tokenized_prompts.astype(jnp.int32)
    nctok = nc_tokenized_prompts.astype(jnp.int32)

    ctx = _sc_gather(ig, ia, global_prompt, attribute_prompt)

    prompts = _build_prompts_call()(ctx, token_prefix, token_suffix)

    nc_prompts, nc_tok, tok = _build_nc_call()(
        nc_token_prefix, global_prompt, attribute_prompt,
        nc_token_suffix, nctok, tokp)

    return (prompts, tok, nc_prompts, nc_tok)


# speedup vs baseline: 1.0215x; 1.0215x over previous
import jax
import jax.numpy as jnp
from jax import lax
from jax.experimental import pallas as pl
from jax.experimental.pallas import tpu as pltpu
from jax.experimental.pallas import tpu_sc as plsc

B = 16
CLS = 100
POOL = 100
CTX_LEN = 12
D = 512
SEQ = 77
SUF = SEQ - 1 - CTX_LEN * 2
NC_SUF = SEQ - 1 - CTX_LEN
NC_SEQ = 1 + 2 * CTX_LEN + NC_SUF

CB = 50
NCB = CLS // CB


def _sc_gather_body(ig, ia, gp, ap, out, idx_v, rows_v, sem):
    c = lax.axis_index("c")
    s = lax.axis_index("s")
    wid = s * 2 + c

    @pl.when(wid == 0)
    def _():
        pltpu.sync_copy(ig, idx_v)
        pltpu.async_copy(gp.at[idx_v], rows_v, sem).wait()
        cps = [pltpu.async_copy(rows_v.at[i], out.at[i // 2, i % 2], sem)
               for i in range(B)]
        for cp in cps:
            cp.wait()

    @pl.when(wid == 1)
    def _():
        pltpu.sync_copy(ia, idx_v)
        pltpu.async_copy(ap.at[idx_v], rows_v, sem).wait()
        cps = [pltpu.async_copy(rows_v.at[i], out.at[8 + i // 2, i % 2], sem)
               for i in range(B)]
        for cp in cps:
            cp.wait()


def _sc_gather(ig, ia, gp, ap):
    return pl.kernel(
        _sc_gather_body,
        out_type=jax.ShapeDtypeStruct((B, 2, CTX_LEN, D), jnp.float32),
        mesh=plsc.VectorSubcoreMesh(core_axis_name="c", subcore_axis_name="s"),
        compiler_params=pltpu.CompilerParams(use_tc_tiling_on_sc=False),
        scratch_types=[
            pltpu.VMEM((B,), jnp.int32),
            pltpu.VMEM((B, CTX_LEN, D), jnp.float32),
            pltpu.SemaphoreType.DMA,
        ],
    )(ig, ia, gp, ap)


NBUF = 3
NSTEP = B * NCB


def _assemble(buf, ctx, pre, suf, cb):
    buf[:, 0:1, :] = pre[...]
    buf[:, 1:1 + CTX_LEN, :] = jnp.broadcast_to(ctx[0, 0], (CB, CTX_LEN, D))
    buf[:, 1 + CTX_LEN:1 + 2 * CTX_LEN, :] = jnp.broadcast_to(
        ctx[0, 1], (CB, CTX_LEN, D))
    buf[:, 1 + 2 * CTX_LEN:SEQ, :] = suf[pl.ds(cb * CB, CB)]


def _body(ctx, pre, suf, ncp, ncs, gp, ap, nctok, tokp,
          out, nc_out, nc_tok_out, tok_out,
          bufs, ncbuf, nctbuf, sems, sem_nc, sem_sm):
    s = pl.program_id(0)
    cb = s % NCB
    i = s % NBUF

    @pl.when(s == 0)
    def _():
        for b in range(B):
            pltpu.make_async_copy(
                tokp, tok_out.at[pl.ds(b * CLS, CLS)], sem_sm).start()
        nctbuf[...] = jnp.broadcast_to(nctok[...], (POOL, SEQ))
        pltpu.make_async_copy(nctbuf, nc_tok_out, sem_sm).start()
        ncbuf[:, 0:1, :] = jnp.broadcast_to(ncp[...], (CB, 1, D))
        ncbuf[:, 1:1 + CTX_LEN, :] = gp[pl.ds(0, CB)]
        ncbuf[:, 1 + CTX_LEN:1 + 2 * CTX_LEN, :] = ap[pl.ds(0, CB)]
        ncbuf[:, 1 + 2 * CTX_LEN:NC_SEQ, :] = jnp.broadcast_to(
            ncs[...], (CB, NC_SUF, D))
        pltpu.make_async_copy(ncbuf, nc_out.at[pl.ds(0, CB)], sem_nc).start()

    @pl.when(s == 1)
    def _():
        pltpu.make_async_copy(ncbuf, nc_out.at[pl.ds(0, CB)], sem_nc).wait()
        ncbuf[:, 0:1, :] = jnp.broadcast_to(ncp[...], (CB, 1, D))
        ncbuf[:, 1:1 + CTX_LEN, :] = gp[pl.ds(CB, CB)]
        ncbuf[:, 1 + CTX_LEN:1 + 2 * CTX_LEN, :] = ap[pl.ds(CB, CB)]
        ncbuf[:, 1 + 2 * CTX_LEN:NC_SEQ, :] = jnp.broadcast_to(
            ncs[...], (CB, NC_SUF, D))
        pltpu.make_async_copy(ncbuf, nc_out.at[pl.ds(CB, CB)], sem_nc).start()

    @pl.when(s >= NBUF)
    def _():
        pltpu.make_async_copy(
            bufs.at[i], out.at[pl.ds((s - NBUF) * CB, CB)], sems.at[i]
        ).wait()

    _assemble(bufs.at[i], ctx, pre, suf, cb)
    pltpu.make_async_copy(
        bufs.at[i], out.at[pl.ds(s * CB, CB)], sems.at[i]).start()

    @pl.when(s == NSTEP - 1)
    def _():
        for k in range(NBUF):
            t = NSTEP - NBUF + k
            pltpu.make_async_copy(
                bufs.at[t % NBUF], out.at[pl.ds(t * CB, CB)],
                sems.at[t % NBUF]).wait()
        pltpu.make_async_copy(ncbuf, nc_out.at[pl.ds(CB, CB)], sem_nc).wait()
        pltpu.make_async_copy(nctbuf, nc_tok_out, sem_sm).wait()
        for b in range(B):
            pltpu.make_async_copy(
                tokp, tok_out.at[pl.ds(b * CLS, CLS)], sem_sm).wait()


def _build_call():
    return pl.pallas_call(
        _body,
        grid=(NSTEP,),
        in_specs=[
            pl.BlockSpec((1, 2, CTX_LEN, D), lambda s: (s // NCB, 0, 0, 0)),
            pl.BlockSpec((CB, 1, D), lambda s: (s % NCB, 0, 0)),
            pl.BlockSpec((CLS, SUF, D), lambda s: (0, 0, 0)),
            pl.BlockSpec((1, 1, D), lambda s: (0, 0, 0)),
            pl.BlockSpec((1, NC_SUF, D), lambda s: (0, 0, 0)),
            pl.BlockSpec((POOL, CTX_LEN, D), lambda s: (0, 0, 0)),
            pl.BlockSpec((POOL, CTX_LEN, D), lambda s: (0, 0, 0)),
            pl.BlockSpec((1, SEQ), lambda s: (0, 0)),
            pl.BlockSpec((CLS, SEQ), lambda s: (0, 0)),
        ],
        out_specs=[
            pl.BlockSpec(memory_space=pltpu.MemorySpace.HBM),
            pl.BlockSpec(memory_space=pltpu.MemorySpace.HBM),
            pl.BlockSpec(memory_space=pltpu.MemorySpace.HBM),
            pl.BlockSpec(memory_space=pltpu.MemorySpace.HBM),
        ],
        out_shape=[
            jax.ShapeDtypeStruct((B * CLS, SEQ, D), jnp.float32),
            jax.ShapeDtypeStruct((POOL, NC_SEQ, D), jnp.float32),
            jax.ShapeDtypeStruct((POOL, SEQ), jnp.int32),
            jax.ShapeDtypeStruct((B * CLS, SEQ), jnp.int32),
        ],
        scratch_shapes=[
            pltpu.VMEM((NBUF, CB, SEQ, D), jnp.float32),
            pltpu.VMEM((CB, NC_SEQ, D), jnp.float32),
            pltpu.VMEM((POOL, SEQ), jnp.int32),
            pltpu.SemaphoreType.DMA((NBUF,)),
            pltpu.SemaphoreType.DMA,
            pltpu.SemaphoreType.DMA,
        ],
    )


def kernel(indices_g, indices_a, global_prompt, attribute_prompt,
           token_prefix, token_suffix, nc_token_prefix, nc_token_suffix,
           tokenized_prompts, nc_tokenized_prompts):
    ig = indices_g.astype(jnp.int32)
    ia = indices_a.astype(jnp.int32)
    tokp = tokenized_prompts.astype(jnp.int32)
    nctok = nc_tokenized_prompts.astype(jnp.int32)

    ctx = _sc_gather(ig, ia, global_prompt, attribute_prompt)

    prompts, nc_prompts, nc_tok, tok = _build_call()(
        ctx, token_prefix, token_suffix,
        nc_token_prefix, nc_token_suffix,
        global_prompt, attribute_prompt, nctok, tokp)

    return (prompts, tok, nc_prompts, nc_tok)
